# baseline (speedup 1.0000x reference)
import functools

import jax
import jax.numpy as jnp
from jax import lax
from jax.experimental import pallas as pl
from jax.experimental.pallas import tpu as pltpu
from jax.experimental.pallas import tpu_sc as plsc

_CHUNK = 128


def _sc_gather_t(table_t, idx2d, n_rows):
    d = table_t.shape[0]
    info = plsc.get_sparse_core_info()
    nc, ns = info.num_cores, info.num_subcores
    nw = nc * ns
    rows_per_w = n_rows // nw
    chunks_per_w = rows_per_w // _CHUNK

    mesh = plsc.VectorSubcoreMesh(core_axis_name="c", subcore_axis_name="s")

    @functools.partial(
        pl.kernel,
        mesh=mesh,
        out_type=jax.ShapeDtypeStruct((d, n_rows), jnp.float32),
        scratch_types=[
            pltpu.VMEM((chunks_per_w, _CHUNK), jnp.int32),
            pltpu.VMEM((d, rows_per_w), jnp.float32),
            pltpu.SemaphoreType.DMA,
            pltpu.SemaphoreType.DMA,
        ],
        compiler_params=pltpu.CompilerParams(
            use_tc_tiling_on_sc=False, skip_device_barrier=True
        ),
    )
    def gather_kernel(table_hbm, idx_hbm, out_hbm, idx_v, rows_v, sem_a, sem_b):
        wid = lax.axis_index("s") * nc + lax.axis_index("c")
        pltpu.sync_copy(idx_hbm.at[pl.ds(wid * chunks_per_w, chunks_per_w)], idx_v)

        def fire(j, sem):
            return [
                pltpu.async_copy(
                    table_hbm.at[k].at[idx_v.at[j]],
                    rows_v.at[k, pl.ds(j * _CHUNK, _CHUNK)],
                    sem,
                )
                for k in range(d)
            ]

        pending = fire(0, sem_a)
        for j in range(1, chunks_per_w):
            nxt = fire(j, sem_b if j % 2 else sem_a)
            for cp in pending:
                cp.wait()
            pending = nxt
        for cp in pending:
            cp.wait()
        pltpu.sync_copy(
            rows_v, out_hbm.at[:, pl.ds(wid * rows_per_w, rows_per_w)]
        )

    return gather_kernel(table_t, idx2d)


def _tc_body(qt_ref, pt_ref, rt_ref, s_ref, at_ref, pos_ref, neg_ref):
    qt = qt_ref[...]
    pt = pt_ref[...]
    rt3 = rt_ref[...]
    rt = rt3.reshape(rt3.shape[0], rt3.shape[1] * rt3.shape[2])
    s = s_ref[...]
    at = at_ref[...]

    def nrm_cols(x):
        n = jnp.sqrt(jnp.sum(x * x, axis=0, keepdims=True))
        return x / jnp.maximum(n, 1e-12)

    bt = nrm_cols(qt)
    pnt = nrm_cols(pt)
    ant = nrm_cols(at)
    et = rt - s * bt
    c = jnp.sum(et * et, axis=0, keepdims=True) + 1.0
    ft = (2.0 * s) * et
    fhat = jnp.concatenate([ft, c], axis=0)
    ahat = jnp.concatenate(
        [ant, jnp.ones((1, ant.shape[1]), jnp.float32)], axis=0
    )
    g = jax.lax.dot_general(
        fhat, ahat, (((0,), (0,)), ((), ())),
        preferred_element_type=jnp.float32,
    )
    gc = jnp.maximum(g, 1e-30)
    neg_ref[...] = -(gc * jax.lax.rsqrt(gc))
    vt = et + s * pnt
    pos_ref[...] = -jnp.sqrt(jnp.sum(vt * vt, axis=0))


def kernel(query_embeds, pos_target_embeds, neg_target_embeds, relation_table,
           r_type, is_head_prediction):
    q_n, d = query_embeds.shape
    n_neg = neg_target_embeds.shape[0]

    idx2d = r_type.reshape(q_n // _CHUNK, _CHUNK)
    rel_t = _sc_gather_t(relation_table.T, idx2d, q_n)

    sgn = jnp.where(is_head_prediction, 1.0, -1.0).astype(jnp.float32)
    sgn = sgn.reshape(1, q_n)

    bq = 1024
    grid = (q_n // bq,)
    pos_d, neg_d = pl.pallas_call(
        _tc_body,
        grid=grid,
        in_specs=[
            pl.BlockSpec((d, bq), lambda i: (0, i)),
            pl.BlockSpec((d, bq), lambda i: (0, i)),
            pl.BlockSpec((d, bq // 128, 128), lambda i: (0, i, 0)),
            pl.BlockSpec((1, bq), lambda i: (0, i)),
            pl.BlockSpec((d, n_neg), lambda i: (0, 0)),
        ],
        out_specs=(
            pl.BlockSpec((bq,), lambda i: (i,)),
            pl.BlockSpec((bq, n_neg), lambda i: (i, 0)),
        ),
        out_shape=(
            jax.ShapeDtypeStruct((q_n,), jnp.float32),
            jax.ShapeDtypeStruct((q_n, n_neg), jnp.float32),
        ),
    )(query_embeds.T, pos_target_embeds.T,
      rel_t.reshape(d, q_n // 128, 128), sgn, neg_target_embeds.T)
    return (pos_d, neg_d)


# speedup vs baseline: 1.0465x; 1.0465x over previous
import functools

import jax
import jax.numpy as jnp
from jax import lax
from jax.experimental import pallas as pl
from jax.experimental.pallas import tpu as pltpu
from jax.experimental.pallas import tpu_sc as plsc

_CHUNK = 128


def _sc_gather_t(table_t, idx2d, n_rows):
    d = table_t.shape[0]
    info = plsc.get_sparse_core_info()
    nc, ns = info.num_cores, info.num_subcores
    nw = nc * ns
    rows_per_w = n_rows // nw
    chunks_per_w = rows_per_w // _CHUNK

    mesh = plsc.VectorSubcoreMesh(core_axis_name="c", subcore_axis_name="s")

    @functools.partial(
        pl.kernel,
        mesh=mesh,
        out_type=jax.ShapeDtypeStruct((d, n_rows), jnp.float32),
        scratch_types=[
            pltpu.VMEM((chunks_per_w, _CHUNK), jnp.int32),
            pltpu.VMEM((d, rows_per_w), jnp.float32),
            pltpu.SemaphoreType.DMA,
            pltpu.SemaphoreType.DMA,
        ],
        compiler_params=pltpu.CompilerParams(
            use_tc_tiling_on_sc=False, skip_device_barrier=True
        ),
    )
    def gather_kernel(table_hbm, idx_hbm, out_hbm, idx_v, rows_v, sem_a, sem_b):
        wid = lax.axis_index("s") * nc + lax.axis_index("c")
        pltpu.sync_copy(idx_hbm.at[pl.ds(wid * chunks_per_w, chunks_per_w)], idx_v)

        def fire(j, sem):
            return [
                pltpu.async_copy(
                    table_hbm.at[k].at[idx_v.at[j]],
                    rows_v.at[k, pl.ds(j * _CHUNK, _CHUNK)],
                    sem,
                )
                for k in range(d)
            ]

        pending = fire(0, sem_a)
        for j in range(1, chunks_per_w):
            nxt = fire(j, sem_b if j % 2 else sem_a)
            for cp in pending:
                cp.wait()
            pending = nxt
        for cp in pending:
            cp.wait()
        pltpu.sync_copy(
            rows_v, out_hbm.at[:, pl.ds(wid * rows_per_w, rows_per_w)]
        )

    return gather_kernel(table_t, idx2d)


def _tc_body(qt_ref, pt_ref, rt_ref, s_ref, at_ref, pos_ref, neg_ref):
    qt = qt_ref[...]
    pt = pt_ref[...]
    rt3 = rt_ref[...]
    rt = rt3.reshape(rt3.shape[0], rt3.shape[1] * rt3.shape[2])
    s = s_ref[...]
    at = at_ref[...]

    def nrm_cols(x):
        n = jnp.sqrt(jnp.sum(x * x, axis=0, keepdims=True))
        return x / jnp.maximum(n, 1e-12)

    bt = nrm_cols(qt)
    pnt = nrm_cols(pt)
    ant = nrm_cols(at)
    et = rt - s * bt
    c = jnp.sum(et * et, axis=0, keepdims=True) + 1.0
    ft = (2.0 * s) * et
    fhat = jnp.concatenate([ft, c], axis=0)
    ahat = jnp.concatenate(
        [ant, jnp.ones((1, ant.shape[1]), jnp.float32)], axis=0
    )
    g = jax.lax.dot_general(
        fhat, ahat, (((0,), (0,)), ((), ())),
        preferred_element_type=jnp.float32,
    )
    gc = jnp.maximum(g, 1e-30)
    neg_ref[...] = -(gc * jax.lax.rsqrt(gc))
    vt = et + s * pnt
    pos_ref[...] = -jnp.sqrt(jnp.sum(vt * vt, axis=0))


def kernel(query_embeds, pos_target_embeds, neg_target_embeds, relation_table,
           r_type, is_head_prediction):
    q_n, d = query_embeds.shape
    n_neg = neg_target_embeds.shape[0]

    idx2d = r_type.reshape(q_n // _CHUNK, _CHUNK)
    rel_t = _sc_gather_t(relation_table.T, idx2d, q_n)

    sgn = jnp.where(is_head_prediction, 1.0, -1.0).astype(jnp.float32)
    sgn = sgn.reshape(1, q_n)

    bq = 2048
    grid = (q_n // bq,)
    pos_d, neg_d = pl.pallas_call(
        _tc_body,
        grid=grid,
        in_specs=[
            pl.BlockSpec((d, bq), lambda i: (0, i)),
            pl.BlockSpec((d, bq), lambda i: (0, i)),
            pl.BlockSpec((d, bq // 128, 128), lambda i: (0, i, 0)),
            pl.BlockSpec((1, bq), lambda i: (0, i)),
            pl.BlockSpec((d, n_neg), lambda i: (0, 0)),
        ],
        out_specs=(
            pl.BlockSpec((bq,), lambda i: (i,)),
            pl.BlockSpec((bq, n_neg), lambda i: (i, 0)),
        ),
        out_shape=(
            jax.ShapeDtypeStruct((q_n,), jnp.float32),
            jax.ShapeDtypeStruct((q_n, n_neg), jnp.float32),
        ),
    )(query_embeds.T, pos_target_embeds.T,
      rel_t.reshape(d, q_n // 128, 128), sgn, neg_target_embeds.T)
    return (pos_d, neg_d)
